# baseline (speedup 1.0000x reference)
import functools

import jax
import jax.numpy as jnp
from jax import lax
from jax.experimental import pallas as pl
from jax.experimental.pallas import tpu as pltpu

B = 64
D = 1024
V = 100000
S_LEN = 16
SOFTCAP = 30.0
VB = 2048
NBLK = 49
VP = NBLK * VB
CW = 128
NC = VP // CW
T = 99
MINF = float('-inf')
BIGI = 1 << 30
TINY = 1.1754943508222875e-38


def _matmul_body(pos_ref, hs_ref, emb_ref, temp_ref, out_ref):
    j = pl.program_id(0)
    hs = hs_ref[0]
    raw = lax.dot_general(
        hs, emb_ref[...], (((1,), (1,)), ((), ())),
        preferred_element_type=jnp.float32)
    l = jnp.tanh(raw / SOFTCAP) * SOFTCAP
    l = l / temp_ref[...]
    col = j * VB + lax.broadcasted_iota(jnp.int32, (B, VB), 1)
    out_ref[...] = jnp.where(col < V, l, -1e30)


def _cumsum_lanes(x, n):
    k = 1
    while k < n:
        pad = jnp.zeros((x.shape[0], k), x.dtype)
        x = x + jnp.concatenate([pad, x[:, :n - k]], axis=1)
        k *= 2
    return x


def _threefry_bits(fi):
    u32 = jnp.uint32
    ks = [u32(0), u32(42), u32(0x1BD11BDA) ^ u32(0) ^ u32(42)]
    x0 = jnp.zeros_like(fi) + ks[0]
    x1 = fi + ks[1]
    rots = ((13, 15, 26, 6), (17, 29, 16, 24))
    for i in range(5):
        for r in rots[i % 2]:
            x0 = x0 + x1
            x1 = (x1 << u32(r)) | (x1 >> u32(32 - r))
            x1 = x0 ^ x1
        x0 = x0 + ks[(i + 1) % 3]
        x1 = x1 + ks[(i + 2) % 3] + u32(i + 1)
    return x0 ^ x1


RB = 16
RG = B // RB


def _sample_body(l3_ref, tps_ref, tks_ref, tok_ref, oh_ref):
    L3 = l3_ref[...]
    cmax = jnp.max(L3, axis=2)
    m = jnp.max(cmax, axis=1, keepdims=True)
    e = jnp.exp(L3 - m[:, :, None])
    s = jnp.sum(jnp.sum(e, axis=2), axis=1, keepdims=True)

    iota_c = lax.broadcasted_iota(jnp.int32, (RB, NC), 1)

    def chunk_step(t, cm):
        gm = jnp.max(cm, axis=1, keepdims=True)
        fi = jnp.min(jnp.where(cm == gm, iota_c, NC + 1), axis=1, keepdims=True)
        oh = iota_c == fi
        oh_ref[:, pl.ds(t, 1), :] = oh.astype(jnp.float32)[:, None, :]
        return jnp.where(oh, MINF, cm)

    lax.fori_loop(0, T, chunk_step, cmax, unroll=9)
    OH = oh_ref[...]
    cand = lax.dot_general(
        OH, L3, (((2,), (1,)), ((0,), (0,))),
        precision=lax.Precision.HIGHEST, preferred_element_type=jnp.float32)
    ciota3 = lax.broadcasted_iota(jnp.int32, (RB, NC, 8), 1).astype(jnp.float32)
    cidsf = lax.dot_general(
        OH, ciota3, (((2,), (1,)), ((0,), (0,))),
        precision=lax.Precision.HIGHEST, preferred_element_type=jnp.float32)[:, :, 0]
    cids = cidsf.astype(jnp.int32)
    vid = (cids[:, :, None] * CW
           + lax.broadcasted_iota(jnp.int32, (RB, T, CW), 2))

    lane = lax.broadcasted_iota(jnp.int32, (RB, T), 1)
    lane128 = lax.broadcasted_iota(jnp.int32, (RB, CW), 1)

    lm0 = jnp.max(cand, axis=2)
    hv0 = jnp.min(jnp.where(cand == lm0[:, :, None], vid, BIGI), axis=2)

    def elem_step(t, carry):
        lm, hv, sv, si = carry
        gm = jnp.max(lm, axis=1, keepdims=True)
        win = jnp.min(jnp.where(lm == gm, hv, BIGI), axis=1, keepdims=True)
        hit = lane == t
        sv = jnp.where(hit, gm, sv)
        si = jnp.where(hit, win, si)
        oh1 = jnp.logical_and(lm == gm, hv == win)
        csel = jnp.max(jnp.where(oh1[:, :, None], cand, MINF), axis=1)
        tcid = jnp.min(jnp.where(oh1, cids, BIGI), axis=1, keepdims=True)
        vrow = tcid * CW + lane128
        alive = jnp.logical_or(
            csel < gm, jnp.logical_and(csel == gm, vrow > win))
        cm2 = jnp.where(alive, csel, MINF)
        nlm = jnp.max(cm2, axis=1, keepdims=True)
        nhv = jnp.min(jnp.where(cm2 == nlm, vrow, BIGI), axis=1, keepdims=True)
        lm = jnp.where(oh1, nlm, lm)
        hv = jnp.where(oh1, nhv, hv)
        return lm, hv, sv, si

    _, _, SV, SI = lax.fori_loop(
        0, T, elem_step,
        (lm0, hv0, jnp.zeros((RB, T), jnp.float32), jnp.zeros((RB, T), jnp.int32)),
        unroll=3)

    p = jnp.exp(SV - m) / s
    ps = _cumsum_lanes(p, T)
    keep = jnp.logical_not(
        jnp.logical_or((ps - p) > tps_ref[...], lane >= tks_ref[...]))
    pm = jnp.where(keep, p, 0.0)
    pf = pm / jnp.sum(pm, axis=1, keepdims=True)
    logp = jnp.log(jnp.maximum(pf, 1e-30))

    row = (pl.program_id(0) * RB
           + lax.broadcasted_iota(jnp.int32, (RB, T), 0))
    bits = _threefry_bits((row * V + SI).astype(jnp.uint32))
    fb = (bits >> jnp.uint32(9)) | jnp.uint32(0x3F800000)
    fl = lax.bitcast_convert_type(fb, jnp.float32) - 1.0
    g = -jnp.log(-jnp.log(jnp.maximum(fl, TINY)))

    score = g + logp
    smax = jnp.max(score, axis=1, keepdims=True)
    tok_ref[...] = jnp.min(jnp.where(score == smax, SI, BIGI),
                           axis=1, keepdims=True)


@jax.jit
def kernel(embedding, hidden_states, output_positions, temperatures, top_ps, top_ks):
    pos = output_positions.astype(jnp.int32)
    grid_spec = pltpu.PrefetchScalarGridSpec(
        num_scalar_prefetch=1,
        grid=(NBLK,),
        in_specs=[
            pl.BlockSpec((1, B, D), lambda j, p: (p[0], 0, 0)),
            pl.BlockSpec((VB, D), lambda j, p: (j, 0)),
            pl.BlockSpec((B, 1), lambda j, p: (0, 0)),
        ],
        out_specs=pl.BlockSpec((B, VB), lambda j, p: (0, j)),
    )
    lp = pl.pallas_call(
        _matmul_body,
        grid_spec=grid_spec,
        out_shape=jax.ShapeDtypeStruct((B, VP), jnp.float32),
    )(pos, hidden_states.transpose(1, 0, 2), embedding, temperatures[:, None])

    logits = lp[:, :V]
    l3 = lp.reshape(B, NC, CW)
    tok = pl.pallas_call(
        _sample_body,
        grid=(RG,),
        in_specs=[
            pl.BlockSpec((RB, NC, CW), lambda i: (i, 0, 0)),
            pl.BlockSpec((RB, 1), lambda i: (i, 0)),
            pl.BlockSpec((RB, 1), lambda i: (i, 0)),
        ],
        out_specs=pl.BlockSpec((RB, 1), lambda i: (i, 0)),
        out_shape=jax.ShapeDtypeStruct((B, 1), jnp.int32),
        scratch_shapes=[pltpu.VMEM((RB, T, NC), jnp.float32)],
    )(l3, top_ps[:, None], top_ks[:, None].astype(jnp.int32))
    return tok[:, 0], logits


# speedup vs baseline: 1.0011x; 1.0011x over previous
import functools

import jax
import jax.numpy as jnp
from jax import lax
from jax.experimental import pallas as pl
from jax.experimental.pallas import tpu as pltpu

B = 64
D = 1024
V = 100000
S_LEN = 16
SOFTCAP = 30.0
VB = 2048
NBLK = 49
VP = NBLK * VB
CW = 128
NC = VP // CW
T = 99
MINF = float('-inf')
BIGI = 1 << 30
TINY = 1.1754943508222875e-38


def _matmul_body(pos_ref, hs_ref, emb_ref, temp_ref, out_ref):
    j = pl.program_id(0)
    hs = hs_ref[0]
    raw = lax.dot_general(
        hs, emb_ref[...], (((1,), (1,)), ((), ())),
        preferred_element_type=jnp.float32)
    l = jnp.tanh(raw / SOFTCAP) * SOFTCAP
    l = l / temp_ref[...]
    col = j * VB + lax.broadcasted_iota(jnp.int32, (B, VB), 1)
    out_ref[...] = jnp.where(col < V, l, -1e30)


def _cumsum_lanes(x, n):
    k = 1
    while k < n:
        pad = jnp.zeros((x.shape[0], k), x.dtype)
        x = x + jnp.concatenate([pad, x[:, :n - k]], axis=1)
        k *= 2
    return x


def _threefry_bits(fi):
    u32 = jnp.uint32
    ks = [u32(0), u32(42), u32(0x1BD11BDA) ^ u32(0) ^ u32(42)]
    x0 = jnp.zeros_like(fi) + ks[0]
    x1 = fi + ks[1]
    rots = ((13, 15, 26, 6), (17, 29, 16, 24))
    for i in range(5):
        for r in rots[i % 2]:
            x0 = x0 + x1
            x1 = (x1 << u32(r)) | (x1 >> u32(32 - r))
            x1 = x0 ^ x1
        x0 = x0 + ks[(i + 1) % 3]
        x1 = x1 + ks[(i + 2) % 3] + u32(i + 1)
    return x0 ^ x1


RB = 16
RG = B // RB


def _sample_body(l3_ref, tps_ref, tks_ref, tok_ref, oh_ref):
    L3 = l3_ref[...]
    cmax = jnp.max(L3, axis=2)
    m = jnp.max(cmax, axis=1, keepdims=True)
    e = jnp.exp(L3 - m[:, :, None])
    s = jnp.sum(jnp.sum(e, axis=2), axis=1, keepdims=True)

    iota_c = lax.broadcasted_iota(jnp.int32, (RB, NC), 1)

    def chunk_step(t, cm):
        gm = jnp.max(cm, axis=1, keepdims=True)
        fi = jnp.min(jnp.where(cm == gm, iota_c, NC + 1), axis=1, keepdims=True)
        oh = iota_c == fi
        oh_ref[:, pl.ds(t, 1), :] = oh.astype(jnp.float32)[:, None, :]
        return jnp.where(oh, MINF, cm)

    lax.fori_loop(0, T, chunk_step, cmax, unroll=9)
    OH = oh_ref[...]
    cand = lax.dot_general(
        OH, L3, (((2,), (1,)), ((0,), (0,))),
        precision=lax.Precision.HIGHEST, preferred_element_type=jnp.float32)
    ciota3 = lax.broadcasted_iota(jnp.int32, (RB, NC, 8), 1).astype(jnp.float32)
    cidsf = lax.dot_general(
        OH, ciota3, (((2,), (1,)), ((0,), (0,))),
        precision=lax.Precision.HIGHEST, preferred_element_type=jnp.float32)[:, :, 0]
    cids = cidsf.astype(jnp.int32)
    vid = (cids[:, :, None] * CW
           + lax.broadcasted_iota(jnp.int32, (RB, T, CW), 2))

    lane = lax.broadcasted_iota(jnp.int32, (RB, T), 1)
    lane128 = lax.broadcasted_iota(jnp.int32, (RB, CW), 1)

    lm0 = jnp.max(cand, axis=2)
    hv0 = jnp.min(jnp.where(cand == lm0[:, :, None], vid, BIGI), axis=2)

    def elem_step(t, carry):
        lm, hv, sv, si = carry
        gm = jnp.max(lm, axis=1, keepdims=True)
        win = jnp.min(jnp.where(lm == gm, hv, BIGI), axis=1, keepdims=True)
        hit = lane == t
        sv = jnp.where(hit, gm, sv)
        si = jnp.where(hit, win, si)
        oh1 = jnp.logical_and(lm == gm, hv == win)
        csel = jnp.max(jnp.where(oh1[:, :, None], cand, MINF), axis=1)
        tcid = jnp.min(jnp.where(oh1, cids, BIGI), axis=1, keepdims=True)
        vrow = tcid * CW + lane128
        alive = jnp.logical_or(
            csel < gm, jnp.logical_and(csel == gm, vrow > win))
        cm2 = jnp.where(alive, csel, MINF)
        nlm = jnp.max(cm2, axis=1, keepdims=True)
        nhv = jnp.min(jnp.where(cm2 == nlm, vrow, BIGI), axis=1, keepdims=True)
        lm = jnp.where(oh1, nlm, lm)
        hv = jnp.where(oh1, nhv, hv)
        return lm, hv, sv, si

    _, _, SV, SI = lax.fori_loop(
        0, T, elem_step,
        (lm0, hv0, jnp.zeros((RB, T), jnp.float32), jnp.zeros((RB, T), jnp.int32)),
        unroll=33)

    p = jnp.exp(SV - m) / s
    ps = _cumsum_lanes(p, T)
    keep = jnp.logical_not(
        jnp.logical_or((ps - p) > tps_ref[...], lane >= tks_ref[...]))
    pm = jnp.where(keep, p, 0.0)
    pf = pm / jnp.sum(pm, axis=1, keepdims=True)
    logp = jnp.log(jnp.maximum(pf, 1e-30))

    row = (pl.program_id(0) * RB
           + lax.broadcasted_iota(jnp.int32, (RB, T), 0))
    bits = _threefry_bits((row * V + SI).astype(jnp.uint32))
    fb = (bits >> jnp.uint32(9)) | jnp.uint32(0x3F800000)
    fl = lax.bitcast_convert_type(fb, jnp.float32) - 1.0
    g = -jnp.log(-jnp.log(jnp.maximum(fl, TINY)))

    score = g + logp
    smax = jnp.max(score, axis=1, keepdims=True)
    tok_ref[...] = jnp.min(jnp.where(score == smax, SI, BIGI),
                           axis=1, keepdims=True)


@jax.jit
def kernel(embedding, hidden_states, output_positions, temperatures, top_ps, top_ks):
    pos = output_positions.astype(jnp.int32)
    grid_spec = pltpu.PrefetchScalarGridSpec(
        num_scalar_prefetch=1,
        grid=(NBLK,),
        in_specs=[
            pl.BlockSpec((1, B, D), lambda j, p: (p[0], 0, 0)),
            pl.BlockSpec((VB, D), lambda j, p: (j, 0)),
            pl.BlockSpec((B, 1), lambda j, p: (0, 0)),
        ],
        out_specs=pl.BlockSpec((B, VB), lambda j, p: (0, j)),
    )
    lp = pl.pallas_call(
        _matmul_body,
        grid_spec=grid_spec,
        out_shape=jax.ShapeDtypeStruct((B, VP), jnp.float32),
    )(pos, hidden_states.transpose(1, 0, 2), embedding, temperatures[:, None])

    logits = lp[:, :V]
    l3 = lp.reshape(B, NC, CW)
    tok = pl.pallas_call(
        _sample_body,
        grid=(RG,),
        in_specs=[
            pl.BlockSpec((RB, NC, CW), lambda i: (i, 0, 0)),
            pl.BlockSpec((RB, 1), lambda i: (i, 0)),
            pl.BlockSpec((RB, 1), lambda i: (i, 0)),
        ],
        out_specs=pl.BlockSpec((RB, 1), lambda i: (i, 0)),
        out_shape=jax.ShapeDtypeStruct((B, 1), jnp.int32),
        scratch_shapes=[pltpu.VMEM((RB, T, NC), jnp.float32)],
    )(l3, top_ps[:, None], top_ks[:, None].astype(jnp.int32))
    return tok[:, 0], logits


# speedup vs baseline: 1.6600x; 1.6581x over previous
import functools

import jax
import jax.numpy as jnp
from jax import lax
from jax.experimental import pallas as pl
from jax.experimental.pallas import tpu as pltpu

B = 64
D = 1024
V = 100000
S_LEN = 16
SOFTCAP = 30.0
VB = 2048
NBLK = 49
VP = NBLK * VB
CW = 128
NC = VP // CW
T = 99
MINF = float('-inf')
BIGI = 1 << 30
TINY = 1.1754943508222875e-38


def _matmul_body(pos_ref, hs_ref, emb_ref, temp_ref, out_ref):
    j = pl.program_id(0)
    hs = hs_ref[0]
    raw = lax.dot_general(
        hs, emb_ref[...], (((1,), (1,)), ((), ())),
        preferred_element_type=jnp.float32)
    l = jnp.tanh(raw / SOFTCAP) * SOFTCAP
    l = l / temp_ref[...]
    col = j * VB + lax.broadcasted_iota(jnp.int32, (B, VB), 1)
    out_ref[...] = jnp.where(col < V, l, -1e30)


def _cumsum_lanes(x, n):
    k = 1
    while k < n:
        pad = jnp.zeros((x.shape[0], k), x.dtype)
        x = x + jnp.concatenate([pad, x[:, :n - k]], axis=1)
        k *= 2
    return x


def _threefry_bits(fi):
    u32 = jnp.uint32
    ks = [u32(0), u32(42), u32(0x1BD11BDA) ^ u32(0) ^ u32(42)]
    x0 = jnp.zeros_like(fi) + ks[0]
    x1 = fi + ks[1]
    rots = ((13, 15, 26, 6), (17, 29, 16, 24))
    for i in range(5):
        for r in rots[i % 2]:
            x0 = x0 + x1
            x1 = (x1 << u32(r)) | (x1 >> u32(32 - r))
            x1 = x0 ^ x1
        x0 = x0 + ks[(i + 1) % 3]
        x1 = x1 + ks[(i + 2) % 3] + u32(i + 1)
    return x0 ^ x1


RB = 16
RG = B // RB


def _sample_body(l3_ref, tps_ref, tks_ref, tok_ref, oh_ref):
    L3 = l3_ref[...]
    cmax = jnp.max(L3, axis=2)
    m = jnp.max(cmax, axis=1, keepdims=True)
    e = jnp.exp(L3 - m[:, :, None])
    s = jnp.sum(jnp.sum(e, axis=2), axis=1, keepdims=True)

    iota_c = lax.broadcasted_iota(jnp.int32, (RB, NC), 1)

    def chunk_step(t, cm):
        gm = jnp.max(cm, axis=1, keepdims=True)
        fi = jnp.min(jnp.where(cm == gm, iota_c, NC + 1), axis=1, keepdims=True)
        oh = iota_c == fi
        oh_ref[:, pl.ds(t, 1), :] = oh.astype(jnp.float32)[:, None, :]
        return jnp.where(oh, MINF, cm)

    lax.fori_loop(0, T, chunk_step, cmax, unroll=9)
    OH = oh_ref[...]
    cand = lax.dot_general(
        OH, L3, (((2,), (1,)), ((0,), (0,))),
        precision=lax.Precision.HIGHEST, preferred_element_type=jnp.float32)
    ciota3 = lax.broadcasted_iota(jnp.int32, (RB, NC, 8), 1).astype(jnp.float32)
    cidsf = lax.dot_general(
        OH, ciota3, (((2,), (1,)), ((0,), (0,))),
        precision=lax.Precision.HIGHEST, preferred_element_type=jnp.float32)[:, :, 0]
    cids = cidsf.astype(jnp.int32)
    vid = (cids[:, :, None] * CW
           + lax.broadcasted_iota(jnp.int32, (RB, T, CW), 2))

    lane = lax.broadcasted_iota(jnp.int32, (RB, T), 1)
    lane128 = lax.broadcasted_iota(jnp.int32, (RB, CW), 1)

    titer = lax.broadcasted_iota(jnp.int32, (RB, T, 1), 1)

    def elem_step(t, carry):
        vals, sv3, si3 = carry
        g1 = jnp.max(vals, axis=2, keepdims=True)
        gm3 = jnp.max(g1, axis=1, keepdims=True)
        w1 = jnp.min(jnp.where(vals == gm3, vid, BIGI), axis=2, keepdims=True)
        win3 = jnp.min(w1, axis=1, keepdims=True)
        hit = titer == t
        sv3 = jnp.where(hit, gm3, sv3)
        si3 = jnp.where(hit, win3, si3)
        vals = jnp.where(vid == win3, MINF, vals)
        return vals, sv3, si3

    _, sv3, si3 = lax.fori_loop(
        0, T, elem_step,
        (cand, jnp.zeros((RB, T, 1), jnp.float32),
         jnp.zeros((RB, T, 1), jnp.int32)),
        unroll=9)
    SV = sv3[:, :, 0]
    SI = si3[:, :, 0]

    p = jnp.exp(SV - m) / s
    ps = _cumsum_lanes(p, T)
    keep = jnp.logical_not(
        jnp.logical_or((ps - p) > tps_ref[...], lane >= tks_ref[...]))
    pm = jnp.where(keep, p, 0.0)
    pf = pm / jnp.sum(pm, axis=1, keepdims=True)
    logp = jnp.log(jnp.maximum(pf, 1e-30))

    row = (pl.program_id(0) * RB
           + lax.broadcasted_iota(jnp.int32, (RB, T), 0))
    bits = _threefry_bits((row * V + SI).astype(jnp.uint32))
    fb = (bits >> jnp.uint32(9)) | jnp.uint32(0x3F800000)
    fl = lax.bitcast_convert_type(fb, jnp.float32) - 1.0
    g = -jnp.log(-jnp.log(jnp.maximum(fl, TINY)))

    score = g + logp
    smax = jnp.max(score, axis=1, keepdims=True)
    tok_ref[...] = jnp.min(jnp.where(score == smax, SI, BIGI),
                           axis=1, keepdims=True)


@jax.jit
def kernel(embedding, hidden_states, output_positions, temperatures, top_ps, top_ks):
    pos = output_positions.astype(jnp.int32)
    grid_spec = pltpu.PrefetchScalarGridSpec(
        num_scalar_prefetch=1,
        grid=(NBLK,),
        in_specs=[
            pl.BlockSpec((1, B, D), lambda j, p: (p[0], 0, 0)),
            pl.BlockSpec((VB, D), lambda j, p: (j, 0)),
            pl.BlockSpec((B, 1), lambda j, p: (0, 0)),
        ],
        out_specs=pl.BlockSpec((B, VB), lambda j, p: (0, j)),
    )
    lp = pl.pallas_call(
        _matmul_body,
        grid_spec=grid_spec,
        out_shape=jax.ShapeDtypeStruct((B, VP), jnp.float32),
    )(pos, hidden_states.transpose(1, 0, 2), embedding, temperatures[:, None])

    logits = lp[:, :V]
    l3 = lp.reshape(B, NC, CW)
    tok = pl.pallas_call(
        _sample_body,
        grid=(RG,),
        in_specs=[
            pl.BlockSpec((RB, NC, CW), lambda i: (i, 0, 0)),
            pl.BlockSpec((RB, 1), lambda i: (i, 0)),
            pl.BlockSpec((RB, 1), lambda i: (i, 0)),
        ],
        out_specs=pl.BlockSpec((RB, 1), lambda i: (i, 0)),
        out_shape=jax.ShapeDtypeStruct((B, 1), jnp.int32),
        scratch_shapes=[pltpu.VMEM((RB, T, NC), jnp.float32)],
    )(l3, top_ps[:, None], top_ks[:, None].astype(jnp.int32))
    return tok[:, 0], logits


# speedup vs baseline: 1.6753x; 1.0092x over previous
import functools

import jax
import jax.numpy as jnp
from jax import lax
from jax.experimental import pallas as pl
from jax.experimental.pallas import tpu as pltpu

B = 64
D = 1024
V = 100000
S_LEN = 16
SOFTCAP = 30.0
VB = 2048
NBLK = 49
VP = NBLK * VB
CW = 128
NC = VP // CW
T = 99
MINF = float('-inf')
BIGI = 1 << 30
TINY = 1.1754943508222875e-38


def _matmul_body(pos_ref, hs_ref, emb_ref, temp_ref, out_ref):
    j = pl.program_id(0)
    hs = hs_ref[0]
    raw = lax.dot_general(
        hs, emb_ref[...], (((1,), (1,)), ((), ())),
        preferred_element_type=jnp.float32)
    l = jnp.tanh(raw / SOFTCAP) * SOFTCAP
    l = l / temp_ref[...]
    col = j * VB + lax.broadcasted_iota(jnp.int32, (B, VB), 1)
    out_ref[...] = jnp.where(col < V, l, -1e30)


def _cumsum_lanes(x, n):
    k = 1
    while k < n:
        pad = jnp.zeros((x.shape[0], k), x.dtype)
        x = x + jnp.concatenate([pad, x[:, :n - k]], axis=1)
        k *= 2
    return x


def _threefry_bits(fi):
    u32 = jnp.uint32
    ks = [u32(0), u32(42), u32(0x1BD11BDA) ^ u32(0) ^ u32(42)]
    x0 = jnp.zeros_like(fi) + ks[0]
    x1 = fi + ks[1]
    rots = ((13, 15, 26, 6), (17, 29, 16, 24))
    for i in range(5):
        for r in rots[i % 2]:
            x0 = x0 + x1
            x1 = (x1 << u32(r)) | (x1 >> u32(32 - r))
            x1 = x0 ^ x1
        x0 = x0 + ks[(i + 1) % 3]
        x1 = x1 + ks[(i + 2) % 3] + u32(i + 1)
    return x0 ^ x1


RB = 16
RG = B // RB


def _sample_body(l3_ref, tps_ref, tks_ref, tok_ref, oh_ref, cand_ref):
    L3 = l3_ref[...]
    cmax = jnp.max(L3, axis=2)
    m = jnp.max(cmax, axis=1, keepdims=True)
    e = jnp.exp(L3 - m[:, :, None])
    s = jnp.sum(jnp.sum(e, axis=2), axis=1, keepdims=True)

    iota_c = lax.broadcasted_iota(jnp.int32, (RB, NC), 1)

    def chunk_step(t, cm):
        gm = jnp.max(cm, axis=1, keepdims=True)
        fi = jnp.min(jnp.where(cm == gm, iota_c, NC + 1), axis=1, keepdims=True)
        oh = iota_c == fi
        oh_ref[:, pl.ds(t, 1), :] = oh.astype(jnp.float32)[:, None, :]
        return jnp.where(oh, MINF, cm)

    lax.fori_loop(0, T, chunk_step, cmax, unroll=9)
    OH = oh_ref[...]
    cand = lax.dot_general(
        OH, L3, (((2,), (1,)), ((0,), (0,))),
        precision=lax.Precision.HIGHEST, preferred_element_type=jnp.float32)
    ciota3 = lax.broadcasted_iota(jnp.int32, (RB, NC, 8), 1).astype(jnp.float32)
    cidsf = lax.dot_general(
        OH, ciota3, (((2,), (1,)), ((0,), (0,))),
        precision=lax.Precision.HIGHEST, preferred_element_type=jnp.float32)[:, :, 0]
    cids = cidsf.astype(jnp.int32)
    vid = (cids[:, :, None] * CW
           + lax.broadcasted_iota(jnp.int32, (RB, T, CW), 2))

    lane = lax.broadcasted_iota(jnp.int32, (RB, T), 1)
    lane128 = lax.broadcasted_iota(jnp.int32, (RB, CW), 1)

    titer = lax.broadcasted_iota(jnp.int32, (RB, T, 1), 1)
    cand_ref[...] = cand

    def elem_step(t, carry):
        sv3, si3 = carry
        vals = cand_ref[...]
        g1 = jnp.max(vals, axis=2, keepdims=True)
        gm3 = jnp.max(g1, axis=1, keepdims=True)
        w1 = jnp.min(jnp.where(vals == gm3, vid, BIGI), axis=2, keepdims=True)
        win3 = jnp.min(w1, axis=1, keepdims=True)
        hit = titer == t
        sv3 = jnp.where(hit, gm3, sv3)
        si3 = jnp.where(hit, win3, si3)
        cand_ref[...] = jnp.where(vid == win3, MINF, vals)
        return sv3, si3

    sv3, si3 = lax.fori_loop(
        0, T, elem_step,
        (jnp.zeros((RB, T, 1), jnp.float32),
         jnp.zeros((RB, T, 1), jnp.int32)),
        unroll=9)
    SV = sv3[:, :, 0]
    SI = si3[:, :, 0]

    p = jnp.exp(SV - m) / s
    ps = _cumsum_lanes(p, T)
    keep = jnp.logical_not(
        jnp.logical_or((ps - p) > tps_ref[...], lane >= tks_ref[...]))
    pm = jnp.where(keep, p, 0.0)
    pf = pm / jnp.sum(pm, axis=1, keepdims=True)
    logp = jnp.log(jnp.maximum(pf, 1e-30))

    row = (pl.program_id(0) * RB
           + lax.broadcasted_iota(jnp.int32, (RB, T), 0))
    bits = _threefry_bits((row * V + SI).astype(jnp.uint32))
    fb = (bits >> jnp.uint32(9)) | jnp.uint32(0x3F800000)
    fl = lax.bitcast_convert_type(fb, jnp.float32) - 1.0
    g = -jnp.log(-jnp.log(jnp.maximum(fl, TINY)))

    score = g + logp
    smax = jnp.max(score, axis=1, keepdims=True)
    tok_ref[...] = jnp.min(jnp.where(score == smax, SI, BIGI),
                           axis=1, keepdims=True)


@jax.jit
def kernel(embedding, hidden_states, output_positions, temperatures, top_ps, top_ks):
    pos = output_positions.astype(jnp.int32)
    grid_spec = pltpu.PrefetchScalarGridSpec(
        num_scalar_prefetch=1,
        grid=(NBLK,),
        in_specs=[
            pl.BlockSpec((1, B, D), lambda j, p: (p[0], 0, 0)),
            pl.BlockSpec((VB, D), lambda j, p: (j, 0)),
            pl.BlockSpec((B, 1), lambda j, p: (0, 0)),
        ],
        out_specs=pl.BlockSpec((B, VB), lambda j, p: (0, j)),
    )
    lp = pl.pallas_call(
        _matmul_body,
        grid_spec=grid_spec,
        out_shape=jax.ShapeDtypeStruct((B, VP), jnp.float32),
    )(pos, hidden_states.transpose(1, 0, 2), embedding, temperatures[:, None])

    logits = lp[:, :V]
    l3 = lp.reshape(B, NC, CW)
    tok = pl.pallas_call(
        _sample_body,
        grid=(RG,),
        in_specs=[
            pl.BlockSpec((RB, NC, CW), lambda i: (i, 0, 0)),
            pl.BlockSpec((RB, 1), lambda i: (i, 0)),
            pl.BlockSpec((RB, 1), lambda i: (i, 0)),
        ],
        out_specs=pl.BlockSpec((RB, 1), lambda i: (i, 0)),
        out_shape=jax.ShapeDtypeStruct((B, 1), jnp.int32),
        scratch_shapes=[pltpu.VMEM((RB, T, NC), jnp.float32),
                        pltpu.VMEM((RB, T, CW), jnp.float32)],
    )(l3, top_ps[:, None], top_ks[:, None].astype(jnp.int32))
    return tok[:, 0], logits
